# Initial kernel scaffold; baseline (speedup 1.0000x reference)
import jax
import jax.numpy as jnp
from jax.experimental import pallas as pl


def kernel(x, edge_attr, params, edge_index):
    raise NotImplementedError("write your pallas kernel here")


# speedup vs baseline: 1.0137x; 1.0137x over previous
import functools

import jax
import jax.numpy as jnp
from jax.experimental import pallas as pl

_N_NODES = 10000
_LATENT = 128
_TILE = 2000


def _mlp_body(n_add, n_mat, has_ln, *refs):
    i = 0
    adds = [refs[i + k][...] for k in range(n_add)]
    i += n_add
    mats = [(refs[i + 2 * k][...], refs[i + 2 * k + 1][...]) for k in range(n_mat)]
    i += 2 * n_mat
    b1, W2, b2, W3, b3 = (refs[i + k][...] for k in range(5))
    i += 5
    if has_ln:
        g, be = refs[i][...], refs[i + 1][...]
        i += 2
    out_ref = refs[i]

    dot = functools.partial(jax.lax.dot_general,
                            dimension_numbers=(((1,), (0,)), ((), ())),
                            preferred_element_type=jnp.float32)
    h = b1
    for a in adds:
        h = h + a
    for m, w in mats:
        h = h + dot(m, w)
    h = jax.nn.relu(h)
    h = jax.nn.relu(dot(h, W2) + b2)
    u = dot(h, W3) + b3
    if has_ln:
        mu = jnp.mean(u, axis=-1, keepdims=True)
        v = jnp.mean((u - mu) ** 2, axis=-1, keepdims=True)
        u = (u - mu) * jax.lax.rsqrt(v + 1e-5) * g + be
    out_ref[...] = u


def _row_spec(tile, d):
    return pl.BlockSpec((tile, d), lambda i: (i, 0))


def _full_spec(shape):
    return pl.BlockSpec(shape, lambda i: (0,) * len(shape))


def _fused_mlp(adds, mats, b1, W2, b2, W3, b3, gb, tile=_TILE):
    n = (adds[0] if adds else mats[0][0]).shape[0]
    assert n % tile == 0, (n, tile)
    d_out = W3.shape[1]
    in_specs = [_row_spec(tile, a.shape[1]) for a in adds]
    args = list(adds)
    for m, w in mats:
        in_specs += [_row_spec(tile, m.shape[1]), _full_spec(w.shape)]
        args += [m, w]
    scalars = [b1.reshape(1, -1), W2, b2.reshape(1, -1), W3, b3.reshape(1, -1)]
    if gb is not None:
        scalars += [gb[0].reshape(1, -1), gb[1].reshape(1, -1)]
    in_specs += [_full_spec(s.shape) for s in scalars]
    args += scalars
    body = functools.partial(_mlp_body, len(adds), len(mats), gb is not None)
    return pl.pallas_call(
        body,
        grid=(n // tile,),
        in_specs=in_specs,
        out_specs=_row_spec(tile, d_out),
        out_shape=jax.ShapeDtypeStruct((n, d_out), jnp.float32),
    )(*args)


def _mlp_ln(p, xs_mats, adds=(), tile=_TILE):
    ps = p['mlp']
    mats = [(x, w) for x, w in xs_mats]
    return _fused_mlp(list(adds), mats, ps[0]['b'], ps[1]['W'], ps[1]['b'],
                      ps[2]['W'], ps[2]['b'], (p['g'], p['be']), tile)


def _project2_body(x_ref, wa_ref, wb_ref, ga_ref, gb_ref):
    dot = functools.partial(jax.lax.dot_general,
                            dimension_numbers=(((1,), (0,)), ((), ())),
                            preferred_element_type=jnp.float32)
    x = x_ref[...]
    ga_ref[...] = dot(x, wa_ref[...])
    gb_ref[...] = dot(x, wb_ref[...])


def _project2(x, wa, wb, tile=_TILE):
    n, d = x.shape
    return pl.pallas_call(
        _project2_body,
        grid=(n // tile,),
        in_specs=[_row_spec(tile, d), _full_spec(wa.shape), _full_spec(wb.shape)],
        out_specs=(_row_spec(tile, _LATENT), _row_spec(tile, _LATENT)),
        out_shape=(jax.ShapeDtypeStruct((n, _LATENT), jnp.float32),
                   jax.ShapeDtypeStruct((n, _LATENT), jnp.float32)),
    )(x, wa, wb)


def kernel(x, edge_attr, params, edge_index):
    s = edge_index[0]
    r = edge_index[1]

    x = _mlp_ln(params['enc_node'], [(x, params['enc_node']['mlp'][0]['W'])])
    e = _mlp_ln(params['enc_edge'], [(edge_attr, params['enc_edge']['mlp'][0]['W'])])

    for i in range(5):
        pe = params['proc'][i]['edge']
        pn = params['proc'][i]['node']
        W1 = pe['mlp'][0]['W']
        Wa, Wb, Wc = W1[:_LATENT], W1[_LATENT:2 * _LATENT], W1[2 * _LATENT:]
        Gs, Gr = _project2(x, Wa, Wb)
        ms = jnp.take(Gs, s, axis=0)
        mr = jnp.take(Gr, r, axis=0)
        ue = _mlp_ln(pe, [(e, Wc)], adds=[ms, mr])
        agg = jax.ops.segment_sum(ue, r, num_segments=_N_NODES)
        Wn = pn['mlp'][0]['W']
        x = _mlp_ln(pn, [(x, Wn[:_LATENT]), (agg, Wn[_LATENT:])])
        e = ue

    pd = params['dec']
    W3 = pd[2]['W']
    W3p = jnp.zeros((_LATENT, _LATENT), jnp.float32).at[:, :W3.shape[1]].set(W3)
    b3p = jnp.zeros((_LATENT,), jnp.float32).at[:W3.shape[1]].set(pd[2]['b'])
    out = _fused_mlp([], [(x, pd[0]['W'])], pd[0]['b'], pd[1]['W'], pd[1]['b'],
                     W3p, b3p, None)
    return out[:, :W3.shape[1]]


# speedup vs baseline: 2.2026x; 2.1729x over previous
import functools

import jax
import jax.numpy as jnp
from jax.experimental import pallas as pl
from jax.experimental.pallas import tpu as pltpu
from jax.experimental.pallas import tpu_sc as plsc

_N_NODES = 10000
_LATENT = 128
_TILE = 2000


def _mlp_body(n_add, n_mat, has_ln, *refs):
    i = 0
    adds = [refs[i + k][...] for k in range(n_add)]
    i += n_add
    mats = [(refs[i + 2 * k][...], refs[i + 2 * k + 1][...]) for k in range(n_mat)]
    i += 2 * n_mat
    b1, W2, b2, W3, b3 = (refs[i + k][...] for k in range(5))
    i += 5
    if has_ln:
        g, be = refs[i][...], refs[i + 1][...]
        i += 2
    out_ref = refs[i]

    dot = functools.partial(jax.lax.dot_general,
                            dimension_numbers=(((1,), (0,)), ((), ())),
                            preferred_element_type=jnp.float32)
    h = b1
    for a in adds:
        h = h + a
    for m, w in mats:
        h = h + dot(m, w)
    h = jax.nn.relu(h)
    h = jax.nn.relu(dot(h, W2) + b2)
    u = dot(h, W3) + b3
    if has_ln:
        mu = jnp.mean(u, axis=-1, keepdims=True)
        v = jnp.mean((u - mu) ** 2, axis=-1, keepdims=True)
        u = (u - mu) * jax.lax.rsqrt(v + 1e-5) * g + be
    out_ref[...] = u


def _row_spec(tile, d):
    return pl.BlockSpec((tile, d), lambda i: (i, 0))


def _full_spec(shape):
    return pl.BlockSpec(shape, lambda i: (0,) * len(shape))


def _fused_mlp(adds, mats, b1, W2, b2, W3, b3, gb, tile=_TILE):
    n = (adds[0] if adds else mats[0][0]).shape[0]
    assert n % tile == 0, (n, tile)
    d_out = W3.shape[1]
    in_specs = [_row_spec(tile, a.shape[1]) for a in adds]
    args = list(adds)
    for m, w in mats:
        in_specs += [_row_spec(tile, m.shape[1]), _full_spec(w.shape)]
        args += [m, w]
    scalars = [b1.reshape(1, -1), W2, b2.reshape(1, -1), W3, b3.reshape(1, -1)]
    if gb is not None:
        scalars += [gb[0].reshape(1, -1), gb[1].reshape(1, -1)]
    in_specs += [_full_spec(s.shape) for s in scalars]
    args += scalars
    body = functools.partial(_mlp_body, len(adds), len(mats), gb is not None)
    return pl.pallas_call(
        body,
        grid=(n // tile,),
        in_specs=in_specs,
        out_specs=_row_spec(tile, d_out),
        out_shape=jax.ShapeDtypeStruct((n, d_out), jnp.float32),
    )(*args)


def _mlp_ln(p, xs_mats, adds=(), tile=_TILE):
    ps = p['mlp']
    mats = [(x, w) for x, w in xs_mats]
    return _fused_mlp(list(adds), mats, ps[0]['b'], ps[1]['W'], ps[1]['b'],
                      ps[2]['W'], ps[2]['b'], (p['g'], p['be']), tile)


def _project2_body(x_ref, wa_ref, wb_ref, ga_ref, gb_ref):
    dot = functools.partial(jax.lax.dot_general,
                            dimension_numbers=(((1,), (0,)), ((), ())),
                            preferred_element_type=jnp.float32)
    x = x_ref[...]
    ga_ref[...] = dot(x, wa_ref[...])
    gb_ref[...] = dot(x, wb_ref[...])


def _project2(x, wa, wb, tile=_TILE):
    n, d = x.shape
    return pl.pallas_call(
        _project2_body,
        grid=(n // tile,),
        in_specs=[_row_spec(tile, d), _full_spec(wa.shape), _full_spec(wb.shape)],
        out_specs=(_row_spec(tile, _LATENT), _row_spec(tile, _LATENT)),
        out_shape=(jax.ShapeDtypeStruct((n, _LATENT), jnp.float32),
                   jax.ShapeDtypeStruct((n, _LATENT), jnp.float32)),
    )(x, wa, wb)


_GATHER_WIN = 128


def _sc_gather2(Gs, Gr, s2, r2):
    n = s2.shape[1]

    @functools.partial(
        pl.kernel,
        out_type=(jax.ShapeDtypeStruct((n, _LATENT), jnp.float32),
                  jax.ShapeDtypeStruct((n, _LATENT), jnp.float32)),
        mesh=plsc.VectorSubcoreMesh(core_axis_name="c", subcore_axis_name="s"),
    )
    def k(gs_hbm, gr_hbm, s_hbm, r_hbm, ms_hbm, mr_hbm):
        def body(s_vmem, r_vmem, ms_vmem, mr_vmem):
            pltpu.sync_copy(gs_hbm.at[s_vmem.at[0]], ms_vmem)
            pltpu.sync_copy(gr_hbm.at[r_vmem.at[0]], mr_vmem)

        pltpu.emit_pipeline(
            body,
            grid=(n // _GATHER_WIN,),
            in_specs=[pl.BlockSpec((1, _GATHER_WIN), lambda i: (0, i)),
                      pl.BlockSpec((1, _GATHER_WIN), lambda i: (0, i))],
            out_specs=[pl.BlockSpec((_GATHER_WIN, _LATENT), lambda i: (i, 0)),
                       pl.BlockSpec((_GATHER_WIN, _LATENT), lambda i: (i, 0))],
            core_axis_name=("c", "s"),
            dimension_semantics=(pltpu.PARALLEL,),
        )(s_hbm, r_hbm, ms_hbm, mr_hbm)

    return k(Gs, Gr, s2, r2)


def kernel(x, edge_attr, params, edge_index):
    s2 = edge_index[0].reshape(1, -1)
    r2 = edge_index[1].reshape(1, -1)
    r = edge_index[1]

    x = _mlp_ln(params['enc_node'], [(x, params['enc_node']['mlp'][0]['W'])])
    e = _mlp_ln(params['enc_edge'], [(edge_attr, params['enc_edge']['mlp'][0]['W'])])

    for i in range(5):
        pe = params['proc'][i]['edge']
        pn = params['proc'][i]['node']
        W1 = pe['mlp'][0]['W']
        Wa, Wb, Wc = W1[:_LATENT], W1[_LATENT:2 * _LATENT], W1[2 * _LATENT:]
        Gs, Gr = _project2(x, Wa, Wb)
        ms, mr = _sc_gather2(Gs, Gr, s2, r2)
        ue = _mlp_ln(pe, [(e, Wc)], adds=[ms, mr])
        agg = jax.ops.segment_sum(ue, r, num_segments=_N_NODES)
        Wn = pn['mlp'][0]['W']
        x = _mlp_ln(pn, [(x, Wn[:_LATENT]), (agg, Wn[_LATENT:])])
        e = ue

    pd = params['dec']
    W3 = pd[2]['W']
    W3p = jnp.zeros((_LATENT, _LATENT), jnp.float32).at[:, :W3.shape[1]].set(W3)
    b3p = jnp.zeros((_LATENT,), jnp.float32).at[:W3.shape[1]].set(pd[2]['b'])
    out = _fused_mlp([], [(x, pd[0]['W'])], pd[0]['b'], pd[1]['W'], pd[1]['b'],
                     W3p, b3p, None)
    return out[:, :W3.shape[1]]


# speedup vs baseline: 2.7983x; 1.2704x over previous
import dataclasses
import functools

import jax
import jax.numpy as jnp
from jax.experimental import pallas as pl
from jax.experimental.pallas import tpu as pltpu
from jax.experimental.pallas import tpu_sc as plsc

_N_NODES = 10000
_N_PAD = 10240
_LATENT = 128
_TILE = 2000


def _dot(a, b):
    return jax.lax.dot_general(a.astype(jnp.bfloat16), b.astype(jnp.bfloat16),
                               dimension_numbers=(((1,), (0,)), ((), ())),
                               preferred_element_type=jnp.float32)


def _mlp_body(group_sizes, has_ln, *refs):
    i = 0
    parts = []
    for gsz in group_sizes:
        p = refs[i][...]
        for k in range(1, gsz):
            p = p + refs[i + k][...]
        parts.append(p)
        i += gsz
    W1, b1, W2, b2, W3, b3 = (refs[i + k][...] for k in range(6))
    i += 6
    if has_ln:
        g, be = refs[i][...], refs[i + 1][...]
        i += 2
    out_ref = refs[i]

    acc = None
    off = 0
    for p in parts:
        w = W1[off:off + p.shape[1]] if len(parts) > 1 else W1
        term = _dot(p, w)
        acc = term if acc is None else acc + term
        off += p.shape[1]
    h = jax.nn.relu(acc + b1)
    h = jax.nn.relu(_dot(h, W2) + b2)
    u = _dot(h, W3) + b3
    if has_ln:
        mu = jnp.mean(u, axis=-1, keepdims=True)
        v = jnp.mean(jnp.abs(u - mu) ** 2, axis=-1, keepdims=True)
        u = (u - mu) / jnp.sqrt(v + 1e-5) * g + be
    out_ref[...] = u


def _row_spec(tile, d):
    return pl.BlockSpec((tile, d), lambda i: (i, 0))


def _full_spec(shape):
    return pl.BlockSpec(shape, lambda i: (0,) * len(shape))


def _fused_mlp(groups, W1, b1, W2, b2, W3, b3, gb, tile=_TILE):
    n = groups[0][0].shape[0]
    assert n % tile == 0, (n, tile)
    d_out = W3.shape[1]
    in_specs, args = [], []
    for grp in groups:
        for a in grp:
            in_specs.append(_row_spec(tile, a.shape[1]))
            args.append(a)
    scalars = [W1, b1.reshape(1, -1), W2, b2.reshape(1, -1), W3,
               b3.reshape(1, -1)]
    if gb is not None:
        scalars += [gb[0].reshape(1, -1), gb[1].reshape(1, -1)]
    in_specs += [_full_spec(s.shape) for s in scalars]
    args += scalars
    body = functools.partial(_mlp_body, tuple(len(g) for g in groups),
                             gb is not None)
    return pl.pallas_call(
        body,
        grid=(n // tile,),
        in_specs=in_specs,
        out_specs=_row_spec(tile, d_out),
        out_shape=jax.ShapeDtypeStruct((n, d_out), jnp.float32),
    )(*args)


def _mlp_ln(p, groups, tile=_TILE):
    ps = p['mlp']
    return _fused_mlp(groups, ps[0]['W'], ps[0]['b'], ps[1]['W'], ps[1]['b'],
                      ps[2]['W'], ps[2]['b'], (p['g'], p['be']), tile)


_GATHER_WIN = 128


def _sc_gather2(Gs, Gr, s2, r2):
    n = s2.shape[1]

    @functools.partial(
        pl.kernel,
        out_type=(jax.ShapeDtypeStruct((n, _LATENT), jnp.float32),
                  jax.ShapeDtypeStruct((n, _LATENT), jnp.float32)),
        mesh=plsc.VectorSubcoreMesh(core_axis_name="c", subcore_axis_name="s"),
    )
    def k(gs_hbm, gr_hbm, s_hbm, r_hbm, ms_hbm, mr_hbm):
        def body(s_vmem, r_vmem, ms_vmem, mr_vmem):
            pltpu.sync_copy(gs_hbm.at[s_vmem.at[0]], ms_vmem)
            pltpu.sync_copy(gr_hbm.at[r_vmem.at[0]], mr_vmem)

        pltpu.emit_pipeline(
            body,
            grid=(n // _GATHER_WIN,),
            in_specs=[pl.BlockSpec((1, _GATHER_WIN), lambda i: (0, i)),
                      pl.BlockSpec((1, _GATHER_WIN), lambda i: (0, i))],
            out_specs=[pl.BlockSpec((_GATHER_WIN, _LATENT), lambda i: (i, 0)),
                       pl.BlockSpec((_GATHER_WIN, _LATENT), lambda i: (i, 0))],
            core_axis_name=("c", "s"),
            dimension_semantics=(pltpu.PARALLEL,),
        )(s_hbm, r_hbm, ms_hbm, mr_hbm)

    return k(Gs, Gr, s2, r2)


def _sc_scatter_add(ue, r2, zeros_nl):
    n = ue.shape[0]
    n_pad = _N_PAD
    rows_per_sub = n_pad // 16

    @functools.partial(
        pl.kernel,
        out_type=jax.ShapeDtypeStruct((2, n_pad, _LATENT), jnp.float32),
        mesh=plsc.VectorSubcoreMesh(core_axis_name="c", subcore_axis_name="s"),
        scratch_types=[pltpu.VMEM_SHARED((n_pad, _LATENT), jnp.float32)],
    )
    def k(ue_hbm, r_hbm, z_hbm, out_hbm, acc):
        cid = jax.lax.axis_index("c")
        sid = jax.lax.axis_index("s")
        row0 = sid * rows_per_sub
        pltpu.sync_copy(z_hbm.at[pl.ds(row0, rows_per_sub)],
                        acc.at[pl.ds(row0, rows_per_sub)])
        plsc.subcore_barrier()

        def body(x_vmem, i_vmem):
            pltpu.sync_copy(x_vmem, acc.at[i_vmem.at[0]], add=True)

        pltpu.emit_pipeline(
            body,
            grid=(n // _GATHER_WIN,),
            in_specs=[pl.BlockSpec((_GATHER_WIN, _LATENT), lambda i: (i, 0)),
                      pl.BlockSpec((1, _GATHER_WIN), lambda i: (0, i))],
            out_specs=[],
            core_axis_name=("c", "s"),
            dimension_semantics=(pltpu.PARALLEL,),
        )(ue_hbm, r_hbm)

        plsc.subcore_barrier()
        pltpu.sync_copy(acc.at[pl.ds(row0, rows_per_sub)],
                        out_hbm.at[cid].at[pl.ds(row0, rows_per_sub)])

    return k(ue, r2, zeros_nl)


_CHUNK = 10080
_SWIN = 96


def _sc_compiler_params():
    cp = pltpu.CompilerParams()
    if "needs_layout_passes" in pltpu.CompilerParams.__dataclass_fields__:
        cp = dataclasses.replace(cp, needs_layout_passes=False)
    return cp


def _sc_seg_sum(ue, perm_p, rs_p, zeros_nl):
    n_pad = _N_PAD
    rows_per_sub = n_pad // 16
    n_win = _CHUNK // _SWIN

    @functools.partial(
        pl.kernel,
        out_type=jax.ShapeDtypeStruct((2, n_pad, _LATENT), jnp.float32),
        mesh=plsc.VectorSubcoreMesh(core_axis_name="c", subcore_axis_name="s"),
        scratch_types=[pltpu.VMEM_SHARED((n_pad, _LATENT), jnp.float32),
                       pltpu.VMEM((_SWIN,), jnp.int32),
                       pltpu.VMEM((_SWIN,), jnp.int32),
                       pltpu.VMEM((_SWIN, _LATENT), jnp.float32),
                       pltpu.VMEM((128, _LATENT), jnp.float32),
                       pltpu.VMEM((1, 128), jnp.int32)],
        compiler_params=_sc_compiler_params(),
    )
    def k(ue_hbm, perm_hbm, rs_hbm, z_hbm, out_hbm, acc, gidx, keys, buf,
          nbuf, idv):
        cid = jax.lax.axis_index("c")
        sid = jax.lax.axis_index("s")
        row0 = sid * rows_per_sub
        pltpu.sync_copy(z_hbm.at[pl.ds(row0, rows_per_sub)],
                        acc.at[pl.ds(row0, rows_per_sub)])
        plsc.subcore_barrier()

        wid = cid * 16 + sid
        dummy = _N_NODES + 4 * wid + 4
        zeros16i = jnp.zeros((16,), jnp.int32)
        lane_iota = jax.lax.iota(jnp.int32, 16)

        def reset_idv():
            @pl.loop(0, 8)
            def _(t):
                idv[0, pl.ds(t * 16, 16)] = zeros16i + dummy

        def flush():
            pltpu.sync_copy(nbuf, acc.at[idv.at[0]], add=True)
            reset_idv()

        reset_idv()

        def append(cur, nptr, acc8):
            for t in range(8):
                nbuf[nptr, pl.ds(t * 16, 16)] = acc8[t]
            idval = jnp.where(cur < 0, zeros16i + dummy, cur)
            plsc.store_scatter(idv.at[0], [zeros16i + nptr], idval,
                               mask=lane_iota == 0)
            nptr = nptr + 1

            def do_flush(p):
                flush()
                return p * 0

            return jax.lax.cond(nptr == 128, do_flush, lambda p: p, nptr)

        def win_body(wi, carry):
            cur, nptr, acc8 = carry
            base = wid * _CHUNK + wi * _SWIN
            pltpu.sync_copy(perm_hbm.at[pl.ds(base, _SWIN)], gidx)
            pltpu.sync_copy(rs_hbm.at[pl.ds(base, _SWIN)], keys)
            pltpu.sync_copy(ue_hbm.at[gidx], buf)

            def row_body(j, carry):
                cur, nptr, acc8 = carry
                key = plsc.load_gather(keys, [zeros16i + j])
                row = tuple(buf[j, pl.ds(t * 16, 16)] for t in range(8))
                is_new = jnp.any(key != cur)

                def on_new(cur, nptr, acc8):
                    nptr = append(cur, nptr, acc8)
                    return key, nptr, row

                def on_cont(cur, nptr, acc8):
                    return cur, nptr, tuple(a + b for a, b in zip(acc8, row))

                return jax.lax.cond(is_new, on_new, on_cont, cur, nptr, acc8)

            return jax.lax.fori_loop(0, _SWIN, row_body, (cur, nptr, acc8))

        acc8_0 = tuple(jnp.zeros((16,), jnp.float32) for _ in range(8))
        cur, nptr, acc8 = jax.lax.fori_loop(
            0, n_win, win_body, (zeros16i - 1, jnp.int32(0), acc8_0))
        nptr = append(cur, nptr, acc8)
        flush()

        plsc.subcore_barrier()
        pltpu.sync_copy(acc.at[pl.ds(row0, rows_per_sub)],
                        out_hbm.at[cid].at[pl.ds(row0, rows_per_sub)])

    return k(ue, perm_p, rs_p, zeros_nl)


def kernel(x, edge_attr, params, edge_index):
    s2 = edge_index[0].reshape(1, -1)
    r2 = edge_index[1].reshape(1, -1)
    zeros_nl = jnp.zeros((_N_PAD, _LATENT), jnp.float32)

    r_idx = edge_index[1]
    perm = jnp.argsort(r_idx, stable=True).astype(jnp.int32)
    rs = jnp.take(r_idx, perm)
    e_pad = 32 * _CHUNK
    n_extra = e_pad - r_idx.shape[0]
    perm_p = jnp.concatenate([perm, jnp.zeros((n_extra,), jnp.int32)])
    rs_p = jnp.concatenate(
        [rs, jnp.full((n_extra,), _N_NODES + 200, jnp.int32)])

    x = _mlp_ln(params['enc_node'], [[x]])
    e = _mlp_ln(params['enc_edge'], [[edge_attr]])

    for i in range(5):
        pe = params['proc'][i]['edge']
        pn = params['proc'][i]['node']
        xs, xr = _sc_gather2(x, x, s2, r2)
        ue = _mlp_ln(pe, [[xs], [xr], [e]])
        agg2 = _sc_seg_sum(ue, perm_p, rs_p, zeros_nl)
        x = _mlp_ln(pn, [[x], [agg2[0, :_N_NODES], agg2[1, :_N_NODES]]])
        e = ue

    pd = params['dec']
    W3 = pd[2]['W']
    W3p = jnp.zeros((_LATENT, _LATENT), jnp.float32).at[:, :W3.shape[1]].set(W3)
    b3p = jnp.zeros((_LATENT,), jnp.float32).at[:W3.shape[1]].set(pd[2]['b'])
    out = _fused_mlp([[x]], pd[0]['W'], pd[0]['b'], pd[1]['W'], pd[1]['b'],
                     W3p, b3p, None)
    return out[:, :W3.shape[1]]
